# Initial kernel scaffold; baseline (speedup 1.0000x reference)
import jax
import jax.numpy as jnp
from jax.experimental import pallas as pl


def kernel(feats, labels, predict, cur, point_queue, cluster_center, anchor_idx):
    raise NotImplementedError("write your pallas kernel here")


# speedup vs baseline: 1.5975x; 1.5975x over previous
import functools

import numpy as np
import jax
import jax.numpy as jnp
from jax import lax
from jax.experimental import pallas as pl
from jax.experimental.pallas import tpu as pltpu
from jax.experimental.pallas import tpu_sc as plsc

_NCLS = 17
_KSUB = 6
_DIM = 64
_PIX = 150
_TEMP = np.float32(0.1)
_NS = 2000
_NP = 2048
_B, _H, _W = 8, 128, 128
_HW = _H * _W
_N = _B * _HW
_QROWS = _NCLS * _KSUB * _PIX
_QPAD = 15360
_HALF = _QPAD // 2
_QT = 1536
_RB = 256

_QS = np.array([0.95, 0.85, 0.75, 0.65, 0.55], dtype=np.float32)
_QIDX = (_QS * np.float32(_N - 1)).astype(np.float32)
_RLO = np.floor(_QIDX).astype(np.int64)
_RHI = np.ceil(_QIDX).astype(np.int64)
_WHI = (_QIDX - _RLO.astype(np.float32)).astype(np.float32)
_WLO = (np.float32(1.0) - _WHI).astype(np.float32)

_SC_MESH = plsc.VectorSubcoreMesh(core_axis_name="c", subcore_axis_name="s")


@functools.partial(
    pl.kernel,
    mesh=_SC_MESH,
    out_type=[
        jax.ShapeDtypeStruct((32, 32, 128), jnp.float32),
        jax.ShapeDtypeStruct((32, 64), jnp.float32),
        jax.ShapeDtypeStruct((32, 64), jnp.int32),
    ],
    scratch_types=[
        pltpu.VMEM((32, 128), jnp.int32),
        pltpu.VMEM((32, 128), jnp.float32),
        pltpu.VMEM((64,), jnp.int32),
        pltpu.VMEM((64,), jnp.float32),
        pltpu.VMEM((64,), jnp.int32),
        pltpu.SemaphoreType.DMA,
    ],
)
def _sc_gather(feat_hbm, idxf_hbm, idxa_hbm, curv_hbm, lab_hbm,
               outf_hbm, outc_hbm, outl_hbm,
               idxf_v, rows_v, idxa_v, c_v, l_v, sem):
    wid = lax.axis_index("s") * 2 + lax.axis_index("c")
    pltpu.sync_copy(idxf_hbm.at[wid], idxf_v)
    pltpu.sync_copy(idxa_hbm.at[wid], idxa_v)
    handles = []
    for t in range(32):
        handles.append(
            pltpu.async_copy(feat_hbm.at[idxf_v.at[t]], rows_v.at[t], sem))
    handles.append(pltpu.async_copy(curv_hbm.at[idxa_v], c_v, sem))
    handles.append(pltpu.async_copy(lab_hbm.at[idxa_v], l_v, sem))
    for h in handles:
        h.wait()
    pltpu.sync_copy(rows_v, outf_hbm.at[wid])
    pltpu.sync_copy(c_v, outc_hbm.at[wid])
    pltpu.sync_copy(l_v, outl_hbm.at[wid])


def _prep_body(curv_ref, xr_ref, ccol_ref, crow_ref, lcol_ref, lrow_ref,
               xn_ref, y_ref, pos_ref, win_ref):
    bits = lax.bitcast_convert_type(curv_ref[...], jnp.int32)

    ths = []
    for k in range(5):
        r_lo = int(_RLO[k])
        r_hi = int(_RHI[k])

        def body(_, lohi, r_lo=r_lo):
            lo, hi = lohi
            mid = lax.div(lo + hi, jnp.int32(2))
            cnt = jnp.sum((bits <= mid).astype(jnp.int32))
            gi = (cnt >= (r_lo + 1)).astype(jnp.int32)
            return (lo + (1 - gi) * (mid + 1 - lo), hi + gi * (mid - hi))

        lo, _ = lax.fori_loop(0, 31, body,
                              (jnp.int32(0), jnp.int32(1 << 30)))
        cle = jnp.sum((bits <= lo).astype(jnp.int32))
        nxt = jnp.min(jnp.where(bits > lo, bits, jnp.int32(0x7FFFFFFF)))
        sel = (cle >= (r_hi + 1)).astype(jnp.int32)
        hi_bits = nxt + sel * (lo - nxt)
        lo_val = lax.bitcast_convert_type(lo, jnp.float32)
        hi_val = lax.bitcast_convert_type(hi_bits, jnp.float32)
        ths.append(lo_val * jnp.float32(_WLO[k]) +
                   hi_val * jnp.float32(_WHI[k]))

    def binning(cv, lv, n, valid):
        sub = jnp.zeros_like(lv)
        for th in ths:
            sub = sub + (cv < th).astype(jnp.int32)
        y = jnp.where(valid, lv * _KSUB + sub, -1)
        pos = jnp.where(valid, y * _PIX + n % _PIX, jnp.int32(1 << 20))
        return y, pos

    n_col = lax.broadcasted_iota(jnp.int32, (_NP, 1), 0)
    n_row = lax.broadcasted_iota(jnp.int32, (1, _NP), 1)
    y_col, pos_col = binning(ccol_ref[...], lcol_ref[...], n_col,
                             n_col < _NS)
    _, pos_row = binning(crow_ref[...], lrow_ref[...], n_row, n_row < _NS)

    wins = []
    for cb in range(_NP // _RB):
        pc = pos_col[cb * _RB:(cb + 1) * _RB, :]
        nc = n_col[cb * _RB:(cb + 1) * _RB, :]
        coll = ((pc == pos_row) & (n_row > nc)).astype(jnp.int32)
        wins.append(1 - jnp.max(coll, axis=1, keepdims=True))
    win_col = jnp.concatenate(wins, axis=0)

    x = xr_ref[...]
    ss = jnp.sum(x * x, axis=1, keepdims=True)
    xnorm = x / (jnp.sqrt(ss) + jnp.float32(1e-12))
    xn_ref[...] = jnp.concatenate(
        [xnorm, jnp.zeros((_NP, 128 - _DIM), jnp.float32)], axis=1)
    y_ref[...] = y_col
    pos_ref[...] = pos_col
    win_ref[...] = win_col


_prep_call = pl.pallas_call(
    _prep_body,
    out_shape=[
        jax.ShapeDtypeStruct((_NP, 128), jnp.float32),
        jax.ShapeDtypeStruct((_NP, 1), jnp.int32),
        jax.ShapeDtypeStruct((_NP, 1), jnp.int32),
        jax.ShapeDtypeStruct((_NP, 1), jnp.int32),
    ],
)


@functools.partial(
    pl.kernel,
    mesh=_SC_MESH,
    out_type=jax.ShapeDtypeStruct((_QPAD, 128), jnp.float32),
    scratch_types=[
        pltpu.VMEM((480, 128), jnp.float32),
        pltpu.VMEM((128, 128), jnp.float32),
        pltpu.VMEM((128,), jnp.int32),
        pltpu.SemaphoreType.DMA,
    ],
)
def _sc_scatter(qpad_hbm, xn_hbm, sidx_hbm, out_hbm, cp_v, xr_v, idx_v, sem):
    c = lax.axis_index("c")
    s = lax.axis_index("s")
    r0 = c * _HALF + s * 480
    pltpu.sync_copy(qpad_hbm.at[pl.ds(r0, 480)], cp_v)
    pltpu.sync_copy(cp_v, out_hbm.at[pl.ds(r0, 480)])
    plsc.subcore_barrier()
    pltpu.sync_copy(xn_hbm.at[pl.ds(s * 128, 128)], xr_v)
    pltpu.sync_copy(sidx_hbm.at[c, s], idx_v)
    pltpu.async_copy(xr_v, out_hbm.at[idx_v], sem).wait()


def _main_body(xn_ref, ycol_ref, yrow_ref, newq_ref, cc_ref, out_ref, acc):
    pid = pl.program_id(0)
    i0 = pid * _RB
    xb = xn_ref[pl.ds(i0, _RB), :]
    yb = ycol_ref[pl.ds(i0, _RB), :]
    y_row = yrow_ref[...]
    rowid = i0 + lax.broadcasted_iota(jnp.int32, (_RB, 1), 0)
    valid_i = (rowid < _NS).astype(jnp.float32)

    dn = (((1,), (1,)), ((), ()))
    adc = lax.dot_general(xb, xn_ref[...], dn,
                          preferred_element_type=jnp.float32) / _TEMP
    col = lax.broadcasted_iota(jnp.int32, (1, _NP), 1)
    vj = (col < _NS)
    vjf = vj.astype(jnp.float32)
    m1 = jnp.max(jnp.where(vj, adc, jnp.float32(-1e30)), axis=1,
                 keepdims=True)
    lg = adc - m1
    el = jnp.exp(lg)
    mask_y = (yb == y_row)
    negraw = jnp.sum(el * (1.0 - mask_y.astype(jnp.float32)) * vjf,
                     axis=1, keepdims=True)

    m = jnp.full((_RB, 1), -1e30, jnp.float32)
    sacc = jnp.zeros((_RB, 1), jnp.float32)
    for t in range(_QPAD // _QT):
        qt = newq_ref[pl.ds(t * _QT, _QT), :]
        bt = lax.dot_general(xb, qt, dn,
                             preferred_element_type=jnp.float32) / _TEMP
        colq = t * _QT + lax.broadcasted_iota(jnp.int32, (1, _QT), 1)
        vq = colq < _QROWS
        btm = jnp.where(vq, bt, jnp.float32(-1e30))
        mnew = jnp.maximum(m, jnp.max(btm, axis=1, keepdims=True))
        keep = (vq & (colq // _PIX != yb)).astype(jnp.float32)
        ex = jnp.exp(btm - mnew) * keep
        sacc = sacc * jnp.exp(m - mnew) + jnp.sum(ex, axis=1, keepdims=True)
        m = mnew
    negq = sacc

    lp = lg - jnp.log(el + negq + negraw)
    jeq = (col == rowid)
    mp = (mask_y & jnp.logical_not(jeq) & vj).astype(jnp.float32) * valid_i
    dsum = jnp.sum(mp, axis=1, keepdims=True)
    mlpp = jnp.sum(mp * lp, axis=1, keepdims=True) / jnp.maximum(dsum, 1.0)
    vld = (dsum > 0).astype(jnp.float32)
    s1 = jnp.sum(mlpp * vld)
    s2 = jnp.sum(vld)

    a2 = lax.dot_general(xb, cc_ref[...], dn,
                         preferred_element_type=jnp.float32) / _TEMP
    col2 = lax.broadcasted_iota(jnp.int32, (1, 128), 1)
    v2 = col2 < (_NCLS * _KSUB)
    v2f = v2.astype(jnp.float32)
    m2 = jnp.max(jnp.where(v2, a2, jnp.float32(-1e30)), axis=1,
                 keepdims=True)
    l2 = a2 - m2
    e2 = jnp.exp(l2)
    mask2 = (yb == col2).astype(jnp.float32)
    neg2 = jnp.sum(e2 * (1.0 - mask2) * v2f, axis=1, keepdims=True)
    lp2 = l2 - jnp.log(e2 + neg2)
    d2 = jnp.sum(mask2, axis=1, keepdims=True)
    ml2 = jnp.sum(mask2 * lp2, axis=1, keepdims=True) / jnp.maximum(d2, 1.0)
    s3 = jnp.sum(ml2 * valid_i)

    @pl.when(pid == 0)
    def _():
        acc[0] = 0.0
        acc[1] = 0.0
        acc[2] = 0.0

    acc[0] = acc[0] + s1
    acc[1] = acc[1] + s2
    acc[2] = acc[2] + s3

    @pl.when(pid == pl.num_programs(0) - 1)
    def _():
        loss_ppc = -_TEMP * acc[0] / jnp.maximum(acc[1], 1.0)
        loss_pcc = -_TEMP * (acc[2] / jnp.float32(_NS))
        out_ref[...] = jnp.full((1, 1), loss_ppc + loss_pcc, jnp.float32)


_main_call = pl.pallas_call(
    _main_body,
    grid=(_NP // _RB,),
    in_specs=[
        pl.BlockSpec((_NP, 128), lambda i: (0, 0)),
        pl.BlockSpec((_NP, 1), lambda i: (0, 0)),
        pl.BlockSpec((1, _NP), lambda i: (0, 0)),
        pl.BlockSpec((_QPAD, 128), lambda i: (0, 0)),
        pl.BlockSpec((128, 128), lambda i: (0, 0)),
    ],
    out_specs=pl.BlockSpec((1, 1), lambda i: (0, 0)),
    out_shape=jax.ShapeDtypeStruct((1, 1), jnp.float32),
    scratch_shapes=[pltpu.SMEM((4,), jnp.float32)],
    compiler_params=pltpu.CompilerParams(
        dimension_semantics=("arbitrary",)),
)


def kernel(feats, labels, predict, cur, point_queue, cluster_center,
           anchor_idx):
    del labels
    feats_flat = feats.reshape(-1)
    curv = cur.reshape(-1)
    labf = predict.reshape(-1).astype(jnp.int32)
    aidx = anchor_idx.astype(jnp.int32)
    apad = jnp.concatenate(
        [aidx, jnp.zeros((_NP - _NS,), jnp.int32)])
    bidx = apad // _HW
    hw = apad % _HW
    base = bidx * (_DIM * _HW) + hw
    widx = (base[:, None] +
            (jnp.arange(_DIM, dtype=jnp.int32) * _HW)[None, :])
    idxf = widx.reshape(32, 32, 128)
    idxa = apad.reshape(32, 64)

    g_f, g_c, g_l = _sc_gather(feats_flat, idxf, idxa, curv, labf)
    xr = g_f.reshape(_NP, _DIM)
    c_all = g_c.reshape(_NP)
    l_all = g_l.reshape(_NP)

    xn, y_col, pos_col, win_col = _prep_call(
        curv.reshape(1024, 128), xr,
        c_all.reshape(_NP, 1), c_all.reshape(1, _NP),
        l_all.reshape(_NP, 1), l_all.reshape(1, _NP))

    nvec = jnp.arange(_NP, dtype=jnp.int32).reshape(_NP, 1)
    ok = (win_col == 1) & (nvec < _NS)
    s0 = jnp.where(ok & (pos_col < _HALF), pos_col, jnp.int32(15308))
    s1 = jnp.where(ok & (pos_col >= _HALF) & (pos_col < _QROWS), pos_col,
                   jnp.int32(15316))
    sidx = jnp.stack([s0.reshape(16, 128), s1.reshape(16, 128)], axis=0)

    qp = jnp.concatenate(
        [point_queue.reshape(_QROWS, _DIM),
         jnp.zeros((_QPAD - _QROWS, _DIM), jnp.float32)], axis=0)
    qpad = jnp.concatenate(
        [qp, jnp.zeros((_QPAD, 128 - _DIM), jnp.float32)], axis=1)
    newq = _sc_scatter(qpad, xn, sidx)

    cc0 = jnp.concatenate(
        [cluster_center.reshape(_NCLS * _KSUB, _DIM),
         jnp.zeros((128 - _NCLS * _KSUB, _DIM), jnp.float32)], axis=0)
    ccp = jnp.concatenate(
        [cc0, jnp.zeros((128, 128 - _DIM), jnp.float32)], axis=1)
    out = _main_call(xn, y_col, y_col.reshape(1, _NP), newq, ccp)
    return out[0, 0]
